# baseline (speedup 1.0000x reference)
import functools

import jax
import jax.numpy as jnp
from jax import lax
from jax.experimental import pallas as pl
from jax.experimental.pallas import tpu as pltpu
from jax.experimental.pallas import tpu_sc as plsc

_NC = 2
_NS = 16
_NW = _NC * _NS


def _silu(x):
    return x / (1.0 + jnp.exp(-x))


def _mlp_tail(h, w2, b2, g, be):
    y = jnp.dot(h, w2, preferred_element_type=jnp.float32) + b2
    mu = jnp.mean(y, axis=-1, keepdims=True)
    var = jnp.mean((y - mu) ** 2, axis=-1, keepdims=True)
    return (y - mu) * lax.rsqrt(var + 1e-5) * g + be


def _pre_body(grid_ref, mesh_ref, wg_ref, wm_ref, gc_ref, mc_ref):
    gc_ref[...] = jnp.dot(grid_ref[...], wg_ref[...],
                          preferred_element_type=jnp.float32)
    mc_ref[...] = jnp.dot(mesh_ref[...], wm_ref[...],
                          preferred_element_type=jnp.float32)


def _precompute(grid_feat, mesh_feat, w_grid, w_mesh):
    n_g, hid = grid_feat.shape
    n_m = mesh_feat.shape[0]
    return pl.pallas_call(
        _pre_body,
        out_shape=(
            jax.ShapeDtypeStruct((n_g, hid), jnp.float32),
            jax.ShapeDtypeStruct((n_m, hid), jnp.float32),
        ),
    )(grid_feat, mesh_feat, w_grid, w_mesh)


def _make_gather(e, hid, chunk):
    epw = e // _NW
    nch = epw // chunk
    quads = nch // 4
    tail = nch - 4 * quads
    mesh = plsc.VectorSubcoreMesh(core_axis_name="c", subcore_axis_name="s")
    slot_scratch = [
        pltpu.VMEM((chunk,), jnp.int32),
        pltpu.VMEM((chunk,), jnp.int32),
        pltpu.VMEM((chunk, hid), jnp.float32),
        pltpu.VMEM((chunk, hid), jnp.float32),
        pltpu.SemaphoreType.DMA,
        pltpu.SemaphoreType.DMA,
        pltpu.SemaphoreType.DMA,
        pltpu.SemaphoreType.DMA,
        pltpu.SemaphoreType.DMA,
        pltpu.VMEM((chunk, hid // 2), jnp.int32),
    ]

    @functools.partial(
        pl.kernel,
        out_type=jax.ShapeDtypeStruct((e, hid // 2), jnp.int32),
        mesh=mesh,
        scratch_types=slot_scratch * 4,
    )
    def gather(gc_hbm, mc_hbm, src_hbm, dst_hbm, el_hbm, *scr):
        wid = lax.axis_index("s") * _NC + lax.axis_index("c")
        slots = tuple(tuple(scr[10 * i:10 * i + 10]) for i in range(4))

        def fire_idx(slot, ci):
            base = wid * epw + ci * chunk
            pltpu.async_copy(src_hbm.at[pl.ds(base, chunk)], slot[0], slot[4])
            pltpu.async_copy(dst_hbm.at[pl.ds(base, chunk)], slot[1], slot[5])

        def wait_idx(slot):
            pltpu.make_async_copy(src_hbm.at[pl.ds(0, chunk)],
                                  slot[0], slot[4]).wait()
            pltpu.make_async_copy(dst_hbm.at[pl.ds(0, chunk)],
                                  slot[1], slot[5]).wait()

        def fire_gather(slot):
            pltpu.async_copy(gc_hbm.at[slot[0]], slot[2], slot[6])
            pltpu.async_copy(mc_hbm.at[slot[1]], slot[3], slot[7])

        def wait_gather(slot):
            pltpu.make_async_copy(gc_hbm.at[slot[0]], slot[2],
                                  slot[6]).wait()
            pltpu.make_async_copy(mc_hbm.at[slot[1]], slot[3],
                                  slot[7]).wait()

        def wait_out(slot):
            pltpu.make_async_copy(slot[9], el_hbm.at[pl.ds(0, chunk)],
                                  slot[8]).wait()

        def fire_out(slot, ci):
            base = wid * epw + ci * chunk
            pltpu.async_copy(slot[9], el_hbm.at[pl.ds(base, chunk)], slot[8])

        def vpu_pack(slot):
            buf_s, buf_d, buf_p = slot[2], slot[3], slot[9]
            half = hid // 2

            def row(r, carry):
                for j in range(half // 16):
                    sl = pl.ds(j * 16, 16)
                    sh = pl.ds(half + j * 16, 16)
                    a = buf_s[r, sl] + buf_d[r, sl]
                    b = buf_s[r, sh] + buf_d[r, sh]
                    au = lax.bitcast_convert_type(a, jnp.uint32)
                    bu = lax.bitcast_convert_type(b, jnp.uint32)
                    lo = (au + jnp.uint32(0x8000)) >> jnp.uint32(16)
                    hi = (bu + jnp.uint32(0x8000)) & jnp.uint32(0xFFFF0000)
                    buf_p[r, sl] = lax.bitcast_convert_type(lo | hi,
                                                            jnp.int32)
                return carry

            lax.fori_loop(0, chunk, row, 0)

        def finish(slot, ci):
            wait_gather(slot)
            vpu_pack(slot)
            fire_out(slot, ci)

        def start(slot, first):
            wait_idx(slot)

            @pl.when(jnp.logical_not(first))
            def _():
                wait_out(slot)
            fire_gather(slot)

        fire_idx(slots[0], 0)
        fire_idx(slots[1], 1)
        fire_idx(slots[2], 2)
        fire_idx(slots[3], 3)
        start(slots[0], True)
        start(slots[1], True)

        def step(m, carry):
            not_last = m + 1 < quads
            start(slots[2], m == 0)
            start(slots[3], m == 0)
            finish(slots[0], 4 * m)
            finish(slots[1], 4 * m + 1)

            @pl.when(not_last)
            def _():
                fire_idx(slots[0], 4 * m + 4)
                fire_idx(slots[1], 4 * m + 5)
            finish(slots[2], 4 * m + 2)
            finish(slots[3], 4 * m + 3)

            @pl.when(not_last)
            def _():
                start(slots[0], False)
                start(slots[1], False)
                fire_idx(slots[2], 4 * m + 6)
                fire_idx(slots[3], 4 * m + 7)
            return carry

        lax.fori_loop(0, quads, step, 0)
        for t in range(tail):
            ci = 4 * quads + t
            slot = slots[t]
            fire_idx(slot, ci)
            wait_idx(slot)
            wait_out(slot)
            fire_gather(slot)
            finish(slot, ci)
        for slot in slots:
            wait_out(slot)

    return gather


def _edge_body(g2m_ref, el_ref, w1_ref, b1_ref, w2_ref, b2_ref,
               g_ref, be_ref, out_ref):
    p = el_ref[...]
    lo = pltpu.unpack_elementwise(p, index=0, packed_dtype=jnp.bfloat16,
                                  unpacked_dtype=jnp.float32)
    hi = pltpu.unpack_elementwise(p, index=1, packed_dtype=jnp.bfloat16,
                                  unpacked_dtype=jnp.float32)
    el = jnp.concatenate([lo, hi], axis=-1)
    x = jnp.dot(g2m_ref[...], w1_ref[...], preferred_element_type=jnp.float32)
    x = x + el + b1_ref[...]
    out_ref[...] = _mlp_tail(_silu(x), w2_ref[...], b2_ref[...],
                             g_ref[...], be_ref[...])


def _edge_mlp(g2m, el, w1a, b1, w2, b2, g, be, block):
    e, hid = g2m.shape
    grid = (e // block,)
    row_spec = pl.BlockSpec((block, hid), lambda i: (i, 0))
    el_spec = pl.BlockSpec((block, hid // 2), lambda i: (i, 0))
    full = lambda shape: pl.BlockSpec(shape, lambda i: (0,) * len(shape))
    return pl.pallas_call(
        _edge_body,
        grid=grid,
        in_specs=[
            row_spec, el_spec,
            full((hid, hid)), full((1, hid)),
            full((hid, hid)), full((1, hid)),
            full((1, hid)), full((1, hid)),
        ],
        out_specs=row_spec,
        out_shape=jax.ShapeDtypeStruct((e, hid), jnp.float32),
    )(g2m, el, w1a, b1, w2, b2, g, be)


def _make_scatter(e, n_mesh, hid, chunk):
    epw = e // _NW
    nch = epw // chunk
    rpt = (n_mesh // _NS) // 8 * 8
    rem = n_mesh - rpt * _NS
    mesh = plsc.VectorSubcoreMesh(core_axis_name="c", subcore_axis_name="s")

    pairs = nch // 2
    slot_scratch = [
        pltpu.VMEM((chunk,), jnp.int32),
        pltpu.VMEM((chunk, hid), jnp.float32),
        pltpu.SemaphoreType.DMA,
        pltpu.SemaphoreType.DMA,
        pltpu.SemaphoreType.DMA,
    ]

    @functools.partial(
        pl.kernel,
        out_type=jax.ShapeDtypeStruct((_NC * n_mesh, hid), jnp.float32),
        mesh=mesh,
        scratch_types=slot_scratch + slot_scratch
        + [pltpu.VMEM_SHARED((n_mesh, hid), jnp.float32)],
    )
    def scatter(ef_hbm, dst_hbm, zeros_hbm, out_hbm,
                idx0, ebuf0, sem_i0, sem_e0, sem_s0,
                idx1, ebuf1, sem_i1, sem_e1, sem_s1, accum):
        cid = lax.axis_index("c")
        sid = lax.axis_index("s")
        wid = sid * _NC + cid
        slots = ((idx0, ebuf0, sem_i0, sem_e0, sem_s0),
                 (idx1, ebuf1, sem_i1, sem_e1, sem_s1))

        def fire_loads(slot, ci):
            base = wid * epw + ci * chunk
            pltpu.async_copy(dst_hbm.at[pl.ds(base, chunk)], slot[0], slot[2])
            pltpu.async_copy(ef_hbm.at[pl.ds(base, chunk)], slot[1], slot[3])

        pltpu.sync_copy(zeros_hbm.at[pl.ds(sid * rpt, rpt)],
                        accum.at[pl.ds(sid * rpt, rpt)])
        if rem:
            @pl.when(sid == 0)
            def _zero_tail():
                pltpu.sync_copy(zeros_hbm.at[pl.ds(_NS * rpt, rem)],
                                accum.at[pl.ds(_NS * rpt, rem)])
        plsc.subcore_barrier()

        fire_loads(slots[0], 0)
        fire_loads(slots[1], 1)

        def step(k, carry):
            for half, slot in enumerate(slots):
                pltpu.make_async_copy(dst_hbm.at[pl.ds(0, chunk)],
                                      slot[0], slot[2]).wait()
                pltpu.make_async_copy(ef_hbm.at[pl.ds(0, chunk)],
                                      slot[1], slot[3]).wait()

                @pl.when(k > 0)
                def _():
                    pltpu.make_async_copy(ebuf0, accum.at[pl.ds(0, chunk)],
                                          slot[4]).wait()

                pltpu.async_copy(slot[1], accum.at[slot[0]], slot[4],
                                 add=True)

            @pl.when(k + 1 < pairs)
            def _():
                fire_loads(slots[0], 2 * k + 2)
                fire_loads(slots[1], 2 * k + 3)
            return carry

        lax.fori_loop(0, pairs, step, 0)
        if nch % 2:
            ci = nch - 1
            slot = slots[0]
            fire_loads(slot, ci)
            pltpu.make_async_copy(dst_hbm.at[pl.ds(0, chunk)],
                                  slot[0], slot[2]).wait()
            pltpu.make_async_copy(ef_hbm.at[pl.ds(0, chunk)],
                                  slot[1], slot[3]).wait()
            pltpu.make_async_copy(ebuf0, accum.at[pl.ds(0, chunk)],
                                  slot[4]).wait()
            pltpu.async_copy(slot[1], accum.at[slot[0]], slot[4], add=True)
        for slot in slots:
            pltpu.make_async_copy(ebuf0, accum.at[pl.ds(0, chunk)],
                                  slot[4]).wait()
        plsc.subcore_barrier()
        pltpu.sync_copy(accum.at[pl.ds(sid * rpt, rpt)],
                        out_hbm.at[pl.ds(cid * n_mesh + sid * rpt, rpt)])
        if rem:
            @pl.when(sid == 0)
            def _copy_tail():
                pltpu.sync_copy(
                    accum.at[pl.ds(_NS * rpt, rem)],
                    out_hbm.at[pl.ds(cid * n_mesh + _NS * rpt, rem)])

    return scatter


def _mesh_body(p_ref, q_ref, mesh_ref,
               dw1a_ref, dw1b_ref, db1_ref, dw2_ref, db2_ref, dg_ref, dbe_ref,
               mesh_out):
    agg = (p_ref[0] + p_ref[1]) + (q_ref[0] + q_ref[1])
    x = (jnp.dot(agg, dw1a_ref[...], preferred_element_type=jnp.float32)
         + jnp.dot(mesh_ref[...], dw1b_ref[...],
                   preferred_element_type=jnp.float32)
         + db1_ref[...])
    mesh_out[...] = mesh_ref[...] + _mlp_tail(
        _silu(x), dw2_ref[...], db2_ref[...], dg_ref[...], dbe_ref[...])


def _final_mesh(partials_a, partials_b, mesh_feat,
                dw1a, dw1b, db1, dw2, db2, dgain, dbe, block):
    n, hid = mesh_feat.shape
    grid = (n // block,)
    row_spec = pl.BlockSpec((block, hid), lambda i: (i, 0))
    full = lambda shape: pl.BlockSpec(shape, lambda i: (0,) * len(shape))
    return pl.pallas_call(
        _mesh_body,
        grid=grid,
        in_specs=[
            pl.BlockSpec((2, block, hid), lambda i: (0, i, 0)),
            pl.BlockSpec((2, block, hid), lambda i: (0, i, 0)),
            row_spec,
            full((hid, hid)), full((hid, hid)), full((1, hid)),
            full((hid, hid)), full((1, hid)), full((1, hid)), full((1, hid)),
        ],
        out_specs=row_spec,
        out_shape=jax.ShapeDtypeStruct((n, hid), jnp.float32),
    )(partials_a, partials_b, mesh_feat,
      dw1a, dw1b, db1, dw2, db2, dgain, dbe)


def _grid_body(grid_ref, sw1_ref, sb1_ref, sw2_ref, sb2_ref, sg_ref, sbe_ref,
               grid_out):
    y = (jnp.dot(grid_ref[...], sw1_ref[...],
                 preferred_element_type=jnp.float32) + sb1_ref[...])
    grid_out[...] = grid_ref[...] + _mlp_tail(
        _silu(y), sw2_ref[...], sb2_ref[...], sg_ref[...], sbe_ref[...])


def _final_grid(grid_feat, sw1, sb1, sw2, sb2, sgain, sbe, block):
    n, hid = grid_feat.shape
    grid = (n // block,)
    row_spec = pl.BlockSpec((block, hid), lambda i: (i, 0))
    full = lambda shape: pl.BlockSpec(shape, lambda i: (0,) * len(shape))
    return pl.pallas_call(
        _grid_body,
        grid=grid,
        in_specs=[
            row_spec,
            full((hid, hid)), full((1, hid)),
            full((hid, hid)), full((1, hid)), full((1, hid)), full((1, hid)),
        ],
        out_specs=row_spec,
        out_shape=jax.ShapeDtypeStruct((n, hid), jnp.float32),
    )(grid_feat, sw1, sb1, sw2, sb2, sgain, sbe)


def kernel(g2m_efeat, grid_feat, mesh_feat, src_idx, dst_idx, num_dst_nodes,
           e_W1, e_b1, e_W2, e_b2, e_g, e_be,
           s_W1, s_b1, s_W2, s_b2, s_g, s_be,
           d_W1, d_b1, d_W2, d_b2, d_g, d_be):
    e, hid = g2m_efeat.shape
    n_mesh = mesh_feat.shape[0]
    chunk = 40

    w1a = e_W1[:hid]
    w_grid = e_W1[hid:2 * hid]
    w_mesh = e_W1[2 * hid:]
    row = lambda v: v.reshape(1, hid)

    grid_c, mesh_c = _precompute(grid_feat, mesh_feat, w_grid, w_mesh)

    eh = e // 2
    src32 = src_idx.astype(jnp.int32)
    dst32 = dst_idx.astype(jnp.int32)
    zeros = jnp.zeros((n_mesh, hid), dtype=jnp.float32)
    gather_fn = _make_gather(eh, hid, chunk)
    scatter_fn = _make_scatter(eh, n_mesh, hid, chunk)

    partials = []
    efeats = []
    for h in range(2):
        sl = slice(h * eh, (h + 1) * eh)
        el = gather_fn(grid_c, mesh_c, src32[sl], dst32[sl])
        efeats.append(
            _edge_mlp(g2m_efeat[sl], el, w1a, row(e_b1), e_W2, row(e_b2),
                      row(e_g), row(e_be), block=8000))
    for h in range(2):
        sl = slice(h * eh, (h + 1) * eh)
        partials.append(
            scatter_fn(efeats[h], dst32[sl], zeros).reshape(_NC, n_mesh, hid))

    grid_new = _final_grid(grid_feat, s_W1, row(s_b1), s_W2, row(s_b2),
                           row(s_g), row(s_be), block=2000)
    mesh_new = _final_mesh(
        partials[0], partials[1], mesh_feat,
        d_W1[:hid], d_W1[hid:], row(d_b1), d_W2, row(d_b2), row(d_g),
        row(d_be), block=2000)

    return grid_new, mesh_new


# speedup vs baseline: 1.0054x; 1.0054x over previous
import functools

import jax
import jax.numpy as jnp
from jax import lax
from jax.experimental import pallas as pl
from jax.experimental.pallas import tpu as pltpu
from jax.experimental.pallas import tpu_sc as plsc

_NC = 2
_NS = 16
_NW = _NC * _NS


def _silu(x):
    return x / (1.0 + jnp.exp(-x))


def _mlp_tail(h, w2, b2, g, be):
    y = jnp.dot(h, w2, preferred_element_type=jnp.float32) + b2
    mu = jnp.mean(y, axis=-1, keepdims=True)
    var = jnp.mean((y - mu) ** 2, axis=-1, keepdims=True)
    return (y - mu) * lax.rsqrt(var + 1e-5) * g + be


def _pre_body(grid_ref, mesh_ref, wg_ref, wm_ref, gc_ref, mc_ref):
    gc_ref[...] = jnp.dot(grid_ref[...], wg_ref[...],
                          preferred_element_type=jnp.float32)
    mc_ref[...] = jnp.dot(mesh_ref[...], wm_ref[...],
                          preferred_element_type=jnp.float32)


def _precompute(grid_feat, mesh_feat, w_grid, w_mesh):
    n_g, hid = grid_feat.shape
    n_m = mesh_feat.shape[0]
    return pl.pallas_call(
        _pre_body,
        out_shape=(
            jax.ShapeDtypeStruct((n_g, hid), jnp.float32),
            jax.ShapeDtypeStruct((n_m, hid), jnp.float32),
        ),
    )(grid_feat, mesh_feat, w_grid, w_mesh)


def _make_gather(e, hid, chunk):
    epw = e // _NW
    nch = epw // chunk
    quads = nch // 4
    tail = nch - 4 * quads
    mesh = plsc.VectorSubcoreMesh(core_axis_name="c", subcore_axis_name="s")
    slot_scratch = [
        pltpu.VMEM((chunk,), jnp.int32),
        pltpu.VMEM((chunk,), jnp.int32),
        pltpu.VMEM((chunk, hid), jnp.float32),
        pltpu.VMEM((chunk, hid), jnp.float32),
        pltpu.SemaphoreType.DMA,
        pltpu.SemaphoreType.DMA,
        pltpu.SemaphoreType.DMA,
        pltpu.SemaphoreType.DMA,
        pltpu.SemaphoreType.DMA,
        pltpu.VMEM((chunk, hid // 2), jnp.int32),
    ]

    @functools.partial(
        pl.kernel,
        out_type=jax.ShapeDtypeStruct((e, hid // 2), jnp.int32),
        mesh=mesh,
        scratch_types=slot_scratch * 4,
    )
    def gather(gc_hbm, mc_hbm, src_hbm, dst_hbm, el_hbm, *scr):
        wid = lax.axis_index("s") * _NC + lax.axis_index("c")
        slots = tuple(tuple(scr[10 * i:10 * i + 10]) for i in range(4))

        def fire_idx(slot, ci):
            base = wid * epw + ci * chunk
            pltpu.async_copy(src_hbm.at[pl.ds(base, chunk)], slot[0], slot[4])
            pltpu.async_copy(dst_hbm.at[pl.ds(base, chunk)], slot[1], slot[5])

        def wait_idx(slot):
            pltpu.make_async_copy(src_hbm.at[pl.ds(0, chunk)],
                                  slot[0], slot[4]).wait()
            pltpu.make_async_copy(dst_hbm.at[pl.ds(0, chunk)],
                                  slot[1], slot[5]).wait()

        def fire_gather(slot):
            pltpu.async_copy(gc_hbm.at[slot[0]], slot[2], slot[6])
            pltpu.async_copy(mc_hbm.at[slot[1]], slot[3], slot[7])

        def wait_gather(slot):
            pltpu.make_async_copy(gc_hbm.at[slot[0]], slot[2],
                                  slot[6]).wait()
            pltpu.make_async_copy(mc_hbm.at[slot[1]], slot[3],
                                  slot[7]).wait()

        def wait_out(slot):
            pltpu.make_async_copy(slot[9], el_hbm.at[pl.ds(0, chunk)],
                                  slot[8]).wait()

        def fire_out(slot, ci):
            base = wid * epw + ci * chunk
            pltpu.async_copy(slot[9], el_hbm.at[pl.ds(base, chunk)], slot[8])

        def vpu_pack(slot):
            buf_s, buf_d, buf_p = slot[2], slot[3], slot[9]
            half = hid // 2

            def row(r, carry):
                for j in range(half // 16):
                    sl = pl.ds(j * 16, 16)
                    sh = pl.ds(half + j * 16, 16)
                    a = buf_s[r, sl] + buf_d[r, sl]
                    b = buf_s[r, sh] + buf_d[r, sh]
                    au = lax.bitcast_convert_type(a, jnp.uint32)
                    bu = lax.bitcast_convert_type(b, jnp.uint32)
                    lo = (au + jnp.uint32(0x8000)) >> jnp.uint32(16)
                    hi = (bu + jnp.uint32(0x8000)) & jnp.uint32(0xFFFF0000)
                    buf_p[r, sl] = lax.bitcast_convert_type(lo | hi,
                                                            jnp.int32)
                return carry

            lax.fori_loop(0, chunk, row, 0)

        def finish(slot, ci):
            wait_gather(slot)
            vpu_pack(slot)
            fire_out(slot, ci)

        def start(slot, first):
            wait_idx(slot)

            @pl.when(jnp.logical_not(first))
            def _():
                wait_out(slot)
            fire_gather(slot)

        fire_idx(slots[0], 0)
        fire_idx(slots[1], 1)
        fire_idx(slots[2], 2)
        fire_idx(slots[3], 3)
        start(slots[0], True)
        start(slots[1], True)

        def step(m, carry):
            not_last = m + 1 < quads
            start(slots[2], m == 0)
            start(slots[3], m == 0)
            finish(slots[0], 4 * m)
            finish(slots[1], 4 * m + 1)

            @pl.when(not_last)
            def _():
                fire_idx(slots[0], 4 * m + 4)
                fire_idx(slots[1], 4 * m + 5)
            finish(slots[2], 4 * m + 2)
            finish(slots[3], 4 * m + 3)

            @pl.when(not_last)
            def _():
                start(slots[0], False)
                start(slots[1], False)
                fire_idx(slots[2], 4 * m + 6)
                fire_idx(slots[3], 4 * m + 7)
            return carry

        lax.fori_loop(0, quads, step, 0)
        for t in range(tail):
            ci = 4 * quads + t
            slot = slots[t]
            fire_idx(slot, ci)
            wait_idx(slot)
            wait_out(slot)
            fire_gather(slot)
            finish(slot, ci)
        for slot in slots:
            wait_out(slot)

    return gather


def _edge_body(g2m_ref, el_ref, w1_ref, b1_ref, w2_ref, b2_ref,
               g_ref, be_ref, out_ref):
    p = el_ref[...]
    lo = pltpu.unpack_elementwise(p, index=0, packed_dtype=jnp.bfloat16,
                                  unpacked_dtype=jnp.float32)
    hi = pltpu.unpack_elementwise(p, index=1, packed_dtype=jnp.bfloat16,
                                  unpacked_dtype=jnp.float32)
    el = jnp.concatenate([lo, hi], axis=-1)
    x = jnp.dot(g2m_ref[...], w1_ref[...], preferred_element_type=jnp.float32)
    x = x + el + b1_ref[...]
    out_ref[...] = _mlp_tail(_silu(x), w2_ref[...], b2_ref[...],
                             g_ref[...], be_ref[...])


def _edge_mlp(g2m, el, w1a, b1, w2, b2, g, be, block):
    e, hid = g2m.shape
    grid = (e // block,)
    row_spec = pl.BlockSpec((block, hid), lambda i: (i, 0))
    el_spec = pl.BlockSpec((block, hid // 2), lambda i: (i, 0))
    full = lambda shape: pl.BlockSpec(shape, lambda i: (0,) * len(shape))
    return pl.pallas_call(
        _edge_body,
        grid=grid,
        in_specs=[
            row_spec, el_spec,
            full((hid, hid)), full((1, hid)),
            full((hid, hid)), full((1, hid)),
            full((1, hid)), full((1, hid)),
        ],
        out_specs=row_spec,
        out_shape=jax.ShapeDtypeStruct((e, hid), jnp.float32),
    )(g2m, el, w1a, b1, w2, b2, g, be)


def _make_scatter(e, n_mesh, hid, chunk):
    epw = e // _NW
    nch = epw // chunk
    rpt = (n_mesh // _NS) // 8 * 8
    rem = n_mesh - rpt * _NS
    mesh = plsc.VectorSubcoreMesh(core_axis_name="c", subcore_axis_name="s")

    pairs = nch // 2
    slot_scratch = [
        pltpu.VMEM((chunk,), jnp.int32),
        pltpu.VMEM((chunk, hid), jnp.float32),
        pltpu.SemaphoreType.DMA,
        pltpu.SemaphoreType.DMA,
        pltpu.SemaphoreType.DMA,
    ]

    @functools.partial(
        pl.kernel,
        out_type=jax.ShapeDtypeStruct((_NC * n_mesh, hid), jnp.float32),
        mesh=mesh,
        scratch_types=slot_scratch + slot_scratch
        + [pltpu.VMEM_SHARED((n_mesh, hid), jnp.float32)],
    )
    def scatter(ef_hbm, dst_hbm, zeros_hbm, out_hbm,
                idx0, ebuf0, sem_i0, sem_e0, sem_s0,
                idx1, ebuf1, sem_i1, sem_e1, sem_s1, accum):
        cid = lax.axis_index("c")
        sid = lax.axis_index("s")
        wid = sid * _NC + cid
        slots = ((idx0, ebuf0, sem_i0, sem_e0, sem_s0),
                 (idx1, ebuf1, sem_i1, sem_e1, sem_s1))

        def fire_loads(slot, ci):
            base = wid * epw + ci * chunk
            pltpu.async_copy(dst_hbm.at[pl.ds(base, chunk)], slot[0], slot[2])
            pltpu.async_copy(ef_hbm.at[pl.ds(base, chunk)], slot[1], slot[3])

        pltpu.sync_copy(zeros_hbm.at[pl.ds(sid * rpt, rpt)],
                        accum.at[pl.ds(sid * rpt, rpt)])
        if rem:
            @pl.when(sid == 0)
            def _zero_tail():
                pltpu.sync_copy(zeros_hbm.at[pl.ds(_NS * rpt, rem)],
                                accum.at[pl.ds(_NS * rpt, rem)])
        plsc.subcore_barrier()

        fire_loads(slots[0], 0)
        fire_loads(slots[1], 1)

        def step(k, carry):
            for half, slot in enumerate(slots):
                pltpu.make_async_copy(dst_hbm.at[pl.ds(0, chunk)],
                                      slot[0], slot[2]).wait()
                pltpu.make_async_copy(ef_hbm.at[pl.ds(0, chunk)],
                                      slot[1], slot[3]).wait()

                @pl.when(k > 0)
                def _():
                    pltpu.make_async_copy(ebuf0, accum.at[pl.ds(0, chunk)],
                                          slot[4]).wait()

                pltpu.async_copy(slot[1], accum.at[slot[0]], slot[4],
                                 add=True)

            @pl.when(k + 1 < pairs)
            def _():
                fire_loads(slots[0], 2 * k + 2)
                fire_loads(slots[1], 2 * k + 3)
            return carry

        lax.fori_loop(0, pairs, step, 0)
        if nch % 2:
            ci = nch - 1
            slot = slots[0]
            fire_loads(slot, ci)
            pltpu.make_async_copy(dst_hbm.at[pl.ds(0, chunk)],
                                  slot[0], slot[2]).wait()
            pltpu.make_async_copy(ef_hbm.at[pl.ds(0, chunk)],
                                  slot[1], slot[3]).wait()
            pltpu.make_async_copy(ebuf0, accum.at[pl.ds(0, chunk)],
                                  slot[4]).wait()
            pltpu.async_copy(slot[1], accum.at[slot[0]], slot[4], add=True)
        for slot in slots:
            pltpu.make_async_copy(ebuf0, accum.at[pl.ds(0, chunk)],
                                  slot[4]).wait()
        plsc.subcore_barrier()
        pltpu.sync_copy(accum.at[pl.ds(sid * rpt, rpt)],
                        out_hbm.at[pl.ds(cid * n_mesh + sid * rpt, rpt)])
        if rem:
            @pl.when(sid == 0)
            def _copy_tail():
                pltpu.sync_copy(
                    accum.at[pl.ds(_NS * rpt, rem)],
                    out_hbm.at[pl.ds(cid * n_mesh + _NS * rpt, rem)])

    return scatter


def _mesh_body(p_ref, q_ref, mesh_ref,
               dw1a_ref, dw1b_ref, db1_ref, dw2_ref, db2_ref, dg_ref, dbe_ref,
               mesh_out):
    agg = (p_ref[0] + p_ref[1]) + (q_ref[0] + q_ref[1])
    x = (jnp.dot(agg, dw1a_ref[...], preferred_element_type=jnp.float32)
         + jnp.dot(mesh_ref[...], dw1b_ref[...],
                   preferred_element_type=jnp.float32)
         + db1_ref[...])
    mesh_out[...] = mesh_ref[...] + _mlp_tail(
        _silu(x), dw2_ref[...], db2_ref[...], dg_ref[...], dbe_ref[...])


def _final_mesh(partials_a, partials_b, mesh_feat,
                dw1a, dw1b, db1, dw2, db2, dgain, dbe, block):
    n, hid = mesh_feat.shape
    grid = (n // block,)
    row_spec = pl.BlockSpec((block, hid), lambda i: (i, 0))
    full = lambda shape: pl.BlockSpec(shape, lambda i: (0,) * len(shape))
    return pl.pallas_call(
        _mesh_body,
        grid=grid,
        in_specs=[
            pl.BlockSpec((2, block, hid), lambda i: (0, i, 0)),
            pl.BlockSpec((2, block, hid), lambda i: (0, i, 0)),
            row_spec,
            full((hid, hid)), full((hid, hid)), full((1, hid)),
            full((hid, hid)), full((1, hid)), full((1, hid)), full((1, hid)),
        ],
        out_specs=row_spec,
        out_shape=jax.ShapeDtypeStruct((n, hid), jnp.float32),
    )(partials_a, partials_b, mesh_feat,
      dw1a, dw1b, db1, dw2, db2, dgain, dbe)


def _grid_body(grid_ref, sw1_ref, sb1_ref, sw2_ref, sb2_ref, sg_ref, sbe_ref,
               grid_out):
    y = (jnp.dot(grid_ref[...], sw1_ref[...],
                 preferred_element_type=jnp.float32) + sb1_ref[...])
    grid_out[...] = grid_ref[...] + _mlp_tail(
        _silu(y), sw2_ref[...], sb2_ref[...], sg_ref[...], sbe_ref[...])


def _final_grid(grid_feat, sw1, sb1, sw2, sb2, sgain, sbe, block):
    n, hid = grid_feat.shape
    grid = (n // block,)
    row_spec = pl.BlockSpec((block, hid), lambda i: (i, 0))
    full = lambda shape: pl.BlockSpec(shape, lambda i: (0,) * len(shape))
    return pl.pallas_call(
        _grid_body,
        grid=grid,
        in_specs=[
            row_spec,
            full((hid, hid)), full((1, hid)),
            full((hid, hid)), full((1, hid)), full((1, hid)), full((1, hid)),
        ],
        out_specs=row_spec,
        out_shape=jax.ShapeDtypeStruct((n, hid), jnp.float32),
    )(grid_feat, sw1, sb1, sw2, sb2, sgain, sbe)


def kernel(g2m_efeat, grid_feat, mesh_feat, src_idx, dst_idx, num_dst_nodes,
           e_W1, e_b1, e_W2, e_b2, e_g, e_be,
           s_W1, s_b1, s_W2, s_b2, s_g, s_be,
           d_W1, d_b1, d_W2, d_b2, d_g, d_be):
    e, hid = g2m_efeat.shape
    n_mesh = mesh_feat.shape[0]
    chunk = 40

    w1a = e_W1[:hid]
    w_grid = e_W1[hid:2 * hid]
    w_mesh = e_W1[2 * hid:]
    row = lambda v: v.reshape(1, hid)

    grid_c, mesh_c = _precompute(grid_feat, mesh_feat, w_grid, w_mesh)

    eh = e // 2
    src32 = src_idx.astype(jnp.int32)
    dst32 = dst_idx.astype(jnp.int32)
    zeros = jnp.zeros((n_mesh, hid), dtype=jnp.float32)
    gather_fn = _make_gather(eh, hid, chunk)
    scatter_fn = _make_scatter(eh, n_mesh, hid, chunk)

    partials = []
    efeats = []
    for h in range(2):
        sl = slice(h * eh, (h + 1) * eh)
        el = gather_fn(grid_c, mesh_c, src32[sl], dst32[sl])
        efeats.append(
            _edge_mlp(g2m_efeat[sl], el, w1a, row(e_b1), e_W2, row(e_b2),
                      row(e_g), row(e_be), block=5000))
    for h in range(2):
        sl = slice(h * eh, (h + 1) * eh)
        partials.append(
            scatter_fn(efeats[h], dst32[sl], zeros).reshape(_NC, n_mesh, hid))

    grid_new = _final_grid(grid_feat, s_W1, row(s_b1), s_W2, row(s_b2),
                           row(s_g), row(s_be), block=2000)
    mesh_new = _final_mesh(
        partials[0], partials[1], mesh_feat,
        d_W1[:hid], d_W1[hid:], row(d_b1), d_W2, row(d_b2), row(d_g),
        row(d_be), block=2000)

    return grid_new, mesh_new


# speedup vs baseline: 1.0109x; 1.0055x over previous
import functools

import jax
import jax.numpy as jnp
from jax import lax
from jax.experimental import pallas as pl
from jax.experimental.pallas import tpu as pltpu
from jax.experimental.pallas import tpu_sc as plsc

_NC = 2
_NS = 16
_NW = _NC * _NS


def _silu(x):
    return x / (1.0 + jnp.exp(-x))


def _mlp_tail(h, w2, b2, g, be):
    y = jnp.dot(h, w2, preferred_element_type=jnp.float32) + b2
    mu = jnp.mean(y, axis=-1, keepdims=True)
    var = jnp.mean((y - mu) ** 2, axis=-1, keepdims=True)
    return (y - mu) * lax.rsqrt(var + 1e-5) * g + be


def _pre_body(grid_ref, mesh_ref, wg_ref, wm_ref, gc_ref, mc_ref):
    gc_ref[...] = jnp.dot(grid_ref[...], wg_ref[...],
                          preferred_element_type=jnp.float32)
    mc_ref[...] = jnp.dot(mesh_ref[...], wm_ref[...],
                          preferred_element_type=jnp.float32)


def _precompute(grid_feat, mesh_feat, w_grid, w_mesh):
    n_g, hid = grid_feat.shape
    n_m = mesh_feat.shape[0]
    return pl.pallas_call(
        _pre_body,
        out_shape=(
            jax.ShapeDtypeStruct((n_g, hid), jnp.float32),
            jax.ShapeDtypeStruct((n_m, hid), jnp.float32),
        ),
    )(grid_feat, mesh_feat, w_grid, w_mesh)


def _make_gather(e, hid, chunk):
    epw = e // _NW
    nch = epw // chunk
    quads = nch // 4
    tail = nch - 4 * quads
    mesh = plsc.VectorSubcoreMesh(core_axis_name="c", subcore_axis_name="s")
    slot_scratch = [
        pltpu.VMEM((chunk,), jnp.int32),
        pltpu.VMEM((chunk,), jnp.int32),
        pltpu.VMEM((chunk, hid), jnp.float32),
        pltpu.VMEM((chunk, hid), jnp.float32),
        pltpu.SemaphoreType.DMA,
        pltpu.SemaphoreType.DMA,
        pltpu.SemaphoreType.DMA,
        pltpu.SemaphoreType.DMA,
        pltpu.SemaphoreType.DMA,
        pltpu.VMEM((chunk, hid // 2), jnp.int32),
    ]

    @functools.partial(
        pl.kernel,
        out_type=jax.ShapeDtypeStruct((e, hid // 2), jnp.int32),
        mesh=mesh,
        scratch_types=slot_scratch * 4,
    )
    def gather(gc_hbm, mc_hbm, src_hbm, dst_hbm, el_hbm, *scr):
        wid = lax.axis_index("s") * _NC + lax.axis_index("c")
        slots = tuple(tuple(scr[10 * i:10 * i + 10]) for i in range(4))

        def fire_idx(slot, ci):
            base = wid * epw + ci * chunk
            pltpu.async_copy(src_hbm.at[pl.ds(base, chunk)], slot[0], slot[4])
            pltpu.async_copy(dst_hbm.at[pl.ds(base, chunk)], slot[1], slot[5])

        def wait_idx(slot):
            pltpu.make_async_copy(src_hbm.at[pl.ds(0, chunk)],
                                  slot[0], slot[4]).wait()
            pltpu.make_async_copy(dst_hbm.at[pl.ds(0, chunk)],
                                  slot[1], slot[5]).wait()

        def fire_gather(slot):
            pltpu.async_copy(gc_hbm.at[slot[0]], slot[2], slot[6])
            pltpu.async_copy(mc_hbm.at[slot[1]], slot[3], slot[7])

        def wait_gather(slot):
            pltpu.make_async_copy(gc_hbm.at[slot[0]], slot[2],
                                  slot[6]).wait()
            pltpu.make_async_copy(mc_hbm.at[slot[1]], slot[3],
                                  slot[7]).wait()

        def wait_out(slot):
            pltpu.make_async_copy(slot[9], el_hbm.at[pl.ds(0, chunk)],
                                  slot[8]).wait()

        def fire_out(slot, ci):
            base = wid * epw + ci * chunk
            pltpu.async_copy(slot[9], el_hbm.at[pl.ds(base, chunk)], slot[8])

        def vpu_pack(slot):
            buf_s, buf_d, buf_p = slot[2], slot[3], slot[9]
            half = hid // 2

            def row(r, carry):
                for j in range(half // 16):
                    sl = pl.ds(j * 16, 16)
                    sh = pl.ds(half + j * 16, 16)
                    a = buf_s[r, sl] + buf_d[r, sl]
                    b = buf_s[r, sh] + buf_d[r, sh]
                    au = lax.bitcast_convert_type(a, jnp.uint32)
                    bu = lax.bitcast_convert_type(b, jnp.uint32)
                    lo = (au + jnp.uint32(0x8000)) >> jnp.uint32(16)
                    hi = (bu + jnp.uint32(0x8000)) & jnp.uint32(0xFFFF0000)
                    buf_p[r, sl] = lax.bitcast_convert_type(lo | hi,
                                                            jnp.int32)
                return carry

            lax.fori_loop(0, chunk, row, 0)

        def finish(slot, ci):
            wait_gather(slot)
            vpu_pack(slot)
            fire_out(slot, ci)

        def start(slot, first):
            wait_idx(slot)

            @pl.when(jnp.logical_not(first))
            def _():
                wait_out(slot)
            fire_gather(slot)

        fire_idx(slots[0], 0)
        fire_idx(slots[1], 1)
        fire_idx(slots[2], 2)
        fire_idx(slots[3], 3)
        start(slots[0], True)
        start(slots[1], True)

        def step(m, carry):
            not_last = m + 1 < quads
            start(slots[2], m == 0)
            start(slots[3], m == 0)
            finish(slots[0], 4 * m)
            finish(slots[1], 4 * m + 1)

            @pl.when(not_last)
            def _():
                fire_idx(slots[0], 4 * m + 4)
                fire_idx(slots[1], 4 * m + 5)
            finish(slots[2], 4 * m + 2)
            finish(slots[3], 4 * m + 3)

            @pl.when(not_last)
            def _():
                start(slots[0], False)
                start(slots[1], False)
                fire_idx(slots[2], 4 * m + 6)
                fire_idx(slots[3], 4 * m + 7)
            return carry

        lax.fori_loop(0, quads, step, 0)
        for t in range(tail):
            ci = 4 * quads + t
            slot = slots[t]
            fire_idx(slot, ci)
            wait_idx(slot)
            wait_out(slot)
            fire_gather(slot)
            finish(slot, ci)
        for slot in slots:
            wait_out(slot)

    return gather


def _edge_body(g2m_ref, el_ref, w1_ref, b1_ref, w2_ref, b2_ref,
               g_ref, be_ref, out_ref):
    p = el_ref[...]
    lo = pltpu.unpack_elementwise(p, index=0, packed_dtype=jnp.bfloat16,
                                  unpacked_dtype=jnp.float32)
    hi = pltpu.unpack_elementwise(p, index=1, packed_dtype=jnp.bfloat16,
                                  unpacked_dtype=jnp.float32)
    el = jnp.concatenate([lo, hi], axis=-1)
    x = jnp.dot(g2m_ref[...], w1_ref[...], preferred_element_type=jnp.float32)
    x = x + el + b1_ref[...]
    out_ref[...] = _mlp_tail(_silu(x), w2_ref[...], b2_ref[...],
                             g_ref[...], be_ref[...])


def _edge_mlp(g2m, el, w1a, b1, w2, b2, g, be, block):
    e, hid = g2m.shape
    grid = (e // block,)
    row_spec = pl.BlockSpec((block, hid), lambda i: (i, 0))
    el_spec = pl.BlockSpec((block, hid // 2), lambda i: (i, 0))
    full = lambda shape: pl.BlockSpec(shape, lambda i: (0,) * len(shape))
    return pl.pallas_call(
        _edge_body,
        grid=grid,
        in_specs=[
            row_spec, el_spec,
            full((hid, hid)), full((1, hid)),
            full((hid, hid)), full((1, hid)),
            full((1, hid)), full((1, hid)),
        ],
        out_specs=row_spec,
        out_shape=jax.ShapeDtypeStruct((e, hid), jnp.float32),
    )(g2m, el, w1a, b1, w2, b2, g, be)


def _make_scatter(e, n_mesh, hid, chunk):
    epw = e // _NW
    nch = epw // chunk
    rpt = (n_mesh // _NS) // 8 * 8
    rem = n_mesh - rpt * _NS
    mesh = plsc.VectorSubcoreMesh(core_axis_name="c", subcore_axis_name="s")

    pairs = nch // 2
    slot_scratch = [
        pltpu.VMEM((chunk,), jnp.int32),
        pltpu.VMEM((chunk, hid), jnp.float32),
        pltpu.SemaphoreType.DMA,
        pltpu.SemaphoreType.DMA,
        pltpu.SemaphoreType.DMA,
    ]

    @functools.partial(
        pl.kernel,
        out_type=jax.ShapeDtypeStruct((_NC * n_mesh, hid), jnp.float32),
        mesh=mesh,
        scratch_types=slot_scratch + slot_scratch
        + [pltpu.VMEM_SHARED((n_mesh, hid), jnp.float32)],
    )
    def scatter(ef_hbm, dst_hbm, zeros_hbm, out_hbm,
                idx0, ebuf0, sem_i0, sem_e0, sem_s0,
                idx1, ebuf1, sem_i1, sem_e1, sem_s1, accum):
        cid = lax.axis_index("c")
        sid = lax.axis_index("s")
        wid = sid * _NC + cid
        slots = ((idx0, ebuf0, sem_i0, sem_e0, sem_s0),
                 (idx1, ebuf1, sem_i1, sem_e1, sem_s1))

        def fire_loads(slot, ci):
            base = wid * epw + ci * chunk
            pltpu.async_copy(dst_hbm.at[pl.ds(base, chunk)], slot[0], slot[2])
            pltpu.async_copy(ef_hbm.at[pl.ds(base, chunk)], slot[1], slot[3])

        pltpu.sync_copy(zeros_hbm.at[pl.ds(sid * rpt, rpt)],
                        accum.at[pl.ds(sid * rpt, rpt)])
        if rem:
            @pl.when(sid == 0)
            def _zero_tail():
                pltpu.sync_copy(zeros_hbm.at[pl.ds(_NS * rpt, rem)],
                                accum.at[pl.ds(_NS * rpt, rem)])
        plsc.subcore_barrier()

        fire_loads(slots[0], 0)
        fire_loads(slots[1], 1)

        def step(k, carry):
            for half, slot in enumerate(slots):
                pltpu.make_async_copy(dst_hbm.at[pl.ds(0, chunk)],
                                      slot[0], slot[2]).wait()
                pltpu.make_async_copy(ef_hbm.at[pl.ds(0, chunk)],
                                      slot[1], slot[3]).wait()

                @pl.when(k > 0)
                def _():
                    pltpu.make_async_copy(ebuf0, accum.at[pl.ds(0, chunk)],
                                          slot[4]).wait()

                pltpu.async_copy(slot[1], accum.at[slot[0]], slot[4],
                                 add=True)

            @pl.when(k + 1 < pairs)
            def _():
                fire_loads(slots[0], 2 * k + 2)
                fire_loads(slots[1], 2 * k + 3)
            return carry

        lax.fori_loop(0, pairs, step, 0)
        if nch % 2:
            ci = nch - 1
            slot = slots[0]
            fire_loads(slot, ci)
            pltpu.make_async_copy(dst_hbm.at[pl.ds(0, chunk)],
                                  slot[0], slot[2]).wait()
            pltpu.make_async_copy(ef_hbm.at[pl.ds(0, chunk)],
                                  slot[1], slot[3]).wait()
            pltpu.make_async_copy(ebuf0, accum.at[pl.ds(0, chunk)],
                                  slot[4]).wait()
            pltpu.async_copy(slot[1], accum.at[slot[0]], slot[4], add=True)
        for slot in slots:
            pltpu.make_async_copy(ebuf0, accum.at[pl.ds(0, chunk)],
                                  slot[4]).wait()
        plsc.subcore_barrier()
        pltpu.sync_copy(accum.at[pl.ds(sid * rpt, rpt)],
                        out_hbm.at[pl.ds(cid * n_mesh + sid * rpt, rpt)])
        if rem:
            @pl.when(sid == 0)
            def _copy_tail():
                pltpu.sync_copy(
                    accum.at[pl.ds(_NS * rpt, rem)],
                    out_hbm.at[pl.ds(cid * n_mesh + _NS * rpt, rem)])

    return scatter


def _mesh_body(p_ref, q_ref, mesh_ref,
               dw1a_ref, dw1b_ref, db1_ref, dw2_ref, db2_ref, dg_ref, dbe_ref,
               mesh_out):
    agg = (p_ref[0] + p_ref[1]) + (q_ref[0] + q_ref[1])
    x = (jnp.dot(agg, dw1a_ref[...], preferred_element_type=jnp.float32)
         + jnp.dot(mesh_ref[...], dw1b_ref[...],
                   preferred_element_type=jnp.float32)
         + db1_ref[...])
    mesh_out[...] = mesh_ref[...] + _mlp_tail(
        _silu(x), dw2_ref[...], db2_ref[...], dg_ref[...], dbe_ref[...])


def _final_mesh(partials_a, partials_b, mesh_feat,
                dw1a, dw1b, db1, dw2, db2, dgain, dbe, block):
    n, hid = mesh_feat.shape
    grid = (n // block,)
    row_spec = pl.BlockSpec((block, hid), lambda i: (i, 0))
    full = lambda shape: pl.BlockSpec(shape, lambda i: (0,) * len(shape))
    return pl.pallas_call(
        _mesh_body,
        grid=grid,
        in_specs=[
            pl.BlockSpec((2, block, hid), lambda i: (0, i, 0)),
            pl.BlockSpec((2, block, hid), lambda i: (0, i, 0)),
            row_spec,
            full((hid, hid)), full((hid, hid)), full((1, hid)),
            full((hid, hid)), full((1, hid)), full((1, hid)), full((1, hid)),
        ],
        out_specs=row_spec,
        out_shape=jax.ShapeDtypeStruct((n, hid), jnp.float32),
    )(partials_a, partials_b, mesh_feat,
      dw1a, dw1b, db1, dw2, db2, dgain, dbe)


def _grid_body(grid_ref, sw1_ref, sb1_ref, sw2_ref, sb2_ref, sg_ref, sbe_ref,
               grid_out):
    y = (jnp.dot(grid_ref[...], sw1_ref[...],
                 preferred_element_type=jnp.float32) + sb1_ref[...])
    grid_out[...] = grid_ref[...] + _mlp_tail(
        _silu(y), sw2_ref[...], sb2_ref[...], sg_ref[...], sbe_ref[...])


def _final_grid(grid_feat, sw1, sb1, sw2, sb2, sgain, sbe, block):
    n, hid = grid_feat.shape
    grid = (n // block,)
    row_spec = pl.BlockSpec((block, hid), lambda i: (i, 0))
    full = lambda shape: pl.BlockSpec(shape, lambda i: (0,) * len(shape))
    return pl.pallas_call(
        _grid_body,
        grid=grid,
        in_specs=[
            row_spec,
            full((hid, hid)), full((1, hid)),
            full((hid, hid)), full((1, hid)), full((1, hid)), full((1, hid)),
        ],
        out_specs=row_spec,
        out_shape=jax.ShapeDtypeStruct((n, hid), jnp.float32),
    )(grid_feat, sw1, sb1, sw2, sb2, sgain, sbe)


def kernel(g2m_efeat, grid_feat, mesh_feat, src_idx, dst_idx, num_dst_nodes,
           e_W1, e_b1, e_W2, e_b2, e_g, e_be,
           s_W1, s_b1, s_W2, s_b2, s_g, s_be,
           d_W1, d_b1, d_W2, d_b2, d_g, d_be):
    e, hid = g2m_efeat.shape
    n_mesh = mesh_feat.shape[0]
    chunk = 40

    w1a = e_W1[:hid]
    w_grid = e_W1[hid:2 * hid]
    w_mesh = e_W1[2 * hid:]
    row = lambda v: v.reshape(1, hid)

    grid_c, mesh_c = _precompute(grid_feat, mesh_feat, w_grid, w_mesh)

    eh = e // 2
    src32 = src_idx.astype(jnp.int32)
    dst32 = dst_idx.astype(jnp.int32)
    zeros = jnp.zeros((n_mesh, hid), dtype=jnp.float32)
    gather_fn = _make_gather(eh, hid, chunk)
    scatter_fn = _make_scatter(eh, n_mesh, hid, chunk)

    partials = []
    efeats = []
    for h in range(2):
        sl = slice(h * eh, (h + 1) * eh)
        el = gather_fn(grid_c, mesh_c, src32[sl], dst32[sl])
        efeats.append(
            _edge_mlp(g2m_efeat[sl], el, w1a, row(e_b1), e_W2, row(e_b2),
                      row(e_g), row(e_be), block=4000))
    for h in range(2):
        sl = slice(h * eh, (h + 1) * eh)
        partials.append(
            scatter_fn(efeats[h], dst32[sl], zeros).reshape(_NC, n_mesh, hid))

    grid_new = _final_grid(grid_feat, s_W1, row(s_b1), s_W2, row(s_b2),
                           row(s_g), row(s_be), block=2000)
    mesh_new = _final_mesh(
        partials[0], partials[1], mesh_feat,
        d_W1[:hid], d_W1[hid:], row(d_b1), d_W2, row(d_b2), row(d_g),
        row(d_be), block=2000)

    return grid_new, mesh_new
